# Initial kernel scaffold; baseline (speedup 1.0000x reference)
import jax
import jax.numpy as jnp
from jax.experimental import pallas as pl


def kernel(x, emb):
    raise NotImplementedError("write your pallas kernel here")


# speedup vs baseline: 2.7491x; 2.7491x over previous
import jax
import jax.numpy as jnp
from jax.experimental import pallas as pl


def _copy_block(emb_ref, out_ref):
    out_ref[...] = emb_ref[...]


def kernel(x, emb):
    seq_len = x.shape[1]
    rows, d = emb.shape
    block_rows = 512
    grid = (seq_len // block_rows,)
    return pl.pallas_call(
        _copy_block,
        grid=grid,
        in_specs=[pl.BlockSpec((block_rows, d), lambda i: (i, 0))],
        out_specs=pl.BlockSpec((block_rows, d), lambda i: (i, 0)),
        out_shape=jax.ShapeDtypeStruct((seq_len, d), emb.dtype),
    )(emb)
